# Initial kernel scaffold; baseline (speedup 1.0000x reference)
import jax
import jax.numpy as jnp
from jax.experimental import pallas as pl


def kernel(x, pos, params, edge_index):
    raise NotImplementedError("write your pallas kernel here")


# speedup vs baseline: 4.7899x; 4.7899x over previous
import functools

import jax
import jax.numpy as jnp
import numpy as np
from jax import lax
from jax.experimental import pallas as pl
from jax.experimental.pallas import tpu as pltpu
from jax.experimental.pallas import tpu_sc as plsc

NN = 10000
NE = 160000
DP = 128
DSH = 16
ADIM = 86
RAD = 5.0
NB = 10
HID = 32
INV_SQRT_ATT = 1.0 / np.sqrt(ADIM)

NC, NS = 2, 16
NW = NC * NS
SUB = 128
NSUB = NE // SUB
NT = -(-NSUB // NW)
ROWS_A = 624
ROWS_B = NN - (NS - 1) * ROWS_A

_mesh = functools.partial(
    plsc.VectorSubcoreMesh, core_axis_name="c", subcore_axis_name="s")


def _sc_gather_multi(tables, idxs):
    ntab = len(tables)
    widths = [t.shape[1] for t in tables]

    @functools.partial(
        pl.kernel,
        mesh=_mesh(),
        out_type=[jax.ShapeDtypeStruct((NE, w), jnp.float32)
                  for w in widths],
        scratch_types=(
            [pltpu.VMEM((SUB,), jnp.int32) for _ in range(ntab)]
            + [pltpu.VMEM((SUB, w), jnp.float32) for w in widths]
            + [pltpu.SemaphoreType.DMA]
        ),
    )
    def k(*refs):
        tab_h = refs[:ntab]
        idx_h = refs[ntab:2 * ntab]
        out_h = refs[2 * ntab:3 * ntab]
        ix = refs[3 * ntab:4 * ntab]
        rr = refs[4 * ntab:5 * ntab]
        sem = refs[5 * ntab]
        w = lax.axis_index("s") * NC + lax.axis_index("c")

        def body(t, carry):
            j = w + t * NW

            @pl.when(j < NSUB)
            def _():
                base = j * SUB
                for n in range(ntab):
                    pltpu.sync_copy(idx_h[n].at[pl.ds(base, SUB)], ix[n])
                for n in range(ntab):
                    pltpu.async_copy(tab_h[n].at[ix[n]], rr[n], sem).wait()
                for n in range(ntab):
                    pltpu.sync_copy(rr[n], out_h[n].at[pl.ds(base, SUB)])

            return carry

        lax.fori_loop(0, NT, body, 0)

    return k(*tables, *idxs)


def _sc_scatter_add(payload, idx, zeros_hbm):
    @functools.partial(
        pl.kernel,
        mesh=_mesh(),
        out_type=jax.ShapeDtypeStruct((NC * NN, DP), jnp.float32),
        scratch_types=[
            pltpu.VMEM((SUB,), jnp.int32),
            pltpu.VMEM((SUB, DP), jnp.float32),
            pltpu.VMEM_SHARED((NN, DP), jnp.float32),
        ],
    )
    def k(pay_h, idx_h, z_h, out_h, ix, pr, acc):
        c = lax.axis_index("c")
        s = lax.axis_index("s")
        w = s * NC + c
        rbase = s * ROWS_A

        @pl.when(s < NS - 1)
        def _():
            pltpu.sync_copy(z_h.at[pl.ds(rbase, ROWS_A)],
                            acc.at[pl.ds(rbase, ROWS_A)])

        @pl.when(s == NS - 1)
        def _():
            pltpu.sync_copy(z_h.at[pl.ds(rbase, ROWS_B)],
                            acc.at[pl.ds(rbase, ROWS_B)])

        plsc.subcore_barrier()

        def body(t, carry):
            j = w + t * NW

            @pl.when(j < NSUB)
            def _():
                base = j * SUB
                pltpu.sync_copy(idx_h.at[pl.ds(base, SUB)], ix)
                pltpu.sync_copy(pay_h.at[pl.ds(base, SUB)], pr)
                pltpu.sync_copy(pr, acc.at[ix], add=True)

            return carry

        lax.fori_loop(0, NT, body, 0)
        plsc.subcore_barrier()

        @pl.when(s < NS - 1)
        def _():
            pltpu.sync_copy(acc.at[pl.ds(rbase, ROWS_A)],
                            out_h.at[pl.ds(c * NN + rbase, ROWS_A)])

        @pl.when(s == NS - 1)
        def _():
            pltpu.sync_copy(acc.at[pl.ds(rbase, ROWS_B)],
                            out_h.at[pl.ds(c * NN + rbase, ROWS_B)])

    return k(payload, idx, zeros_hbm)


_BN = 1000
_BE = 2000


def _tc_node(h, wq, wk, wv):
    def body(h_ref, wq_ref, wk_ref, wv_ref, q_ref, k_ref, v_ref):
        hb = h_ref[...]
        q_ref[...] = jnp.dot(hb, wq_ref[...],
                             preferred_element_type=jnp.float32)
        k_ref[...] = jnp.dot(hb, wk_ref[...],
                             preferred_element_type=jnp.float32)
        v_ref[...] = jnp.dot(hb, wv_ref[...],
                             preferred_element_type=jnp.float32)

    return pl.pallas_call(
        body,
        grid=(NN // _BN,),
        in_specs=[
            pl.BlockSpec((_BN, DP), lambda i: (i, 0)),
            pl.BlockSpec((DP, DP), lambda i: (0, 0)),
            pl.BlockSpec((DP, DP), lambda i: (0, 0)),
            pl.BlockSpec((DP, DP), lambda i: (0, 0)),
        ],
        out_specs=[
            pl.BlockSpec((_BN, DP), lambda i: (i, 0)),
            pl.BlockSpec((_BN, DP), lambda i: (i, 0)),
            pl.BlockSpec((_BN, DP), lambda i: (i, 0)),
        ],
        out_shape=[
            jax.ShapeDtypeStruct((NN, DP), jnp.float32),
            jax.ShapeDtypeStruct((NN, DP), jnp.float32),
            jax.ShapeDtypeStruct((NN, DP), jnp.float32),
        ],
    )(h, wq, wk, wv)


def _tc_geom(pos_src, pos_dst):
    def body(ps_ref, pd_ref, sh_ref, rbf_ref):
        rel = pd_ref[...] - ps_ref[...]
        xx = rel[:, 0:1]
        yy = rel[:, 1:2]
        zz = rel[:, 2:3]
        d = jnp.sqrt(xx * xx + yy * yy + zz * zz)
        inv = 1.0 / (d + 1e-9)
        x = xx * inv
        y = yy * inv
        z = zz * inv
        one = jnp.ones_like(x)
        sh = jnp.concatenate(
            [one, x, y, z,
             x * y, y * z, 0.5 * (3.0 * z * z - 1.0), x * z,
             0.5 * (x * x - y * y),
             y * (3.0 * x * x - y * y), x * y * z, y * (5.0 * z * z - 1.0),
             0.5 * z * (5.0 * z * z - 3.0), x * (5.0 * z * z - 1.0),
             z * (x * x - y * y), x * (x * x - 3.0 * y * y)], axis=1)
        sh_ref[...] = sh
        coli = lax.broadcasted_iota(jnp.int32, (_BE, DSH), 1)
        centers = coli.astype(jnp.float32) * (RAD / (NB - 1))
        rb = jnp.exp(-jnp.square((d - centers) / (RAD / NB)))
        mask = coli < NB
        rbf_ref[...] = jnp.where(mask, rb, 0.0)

    return pl.pallas_call(
        body,
        grid=(NE // _BE,),
        in_specs=[
            pl.BlockSpec((_BE, DP), lambda i: (i, 0)),
            pl.BlockSpec((_BE, DP), lambda i: (i, 0)),
        ],
        out_specs=[
            pl.BlockSpec((_BE, DSH), lambda i: (i, 0)),
            pl.BlockSpec((_BE, DSH), lambda i: (i, 0)),
        ],
        out_shape=[
            jax.ShapeDtypeStruct((NE, DSH), jnp.float32),
            jax.ShapeDtypeStruct((NE, DSH), jnp.float32),
        ],
    )(pos_src, pos_dst)


def _tc_edge(qd, ks, vs, sh, rbf, skp, svp, r1p, b1, rkp, rvp):
    def body(qd_ref, k_ref, v_ref, sh_ref, rbf_ref,
             sk_ref, sv_ref, r1_ref, b1_ref, rk_ref, rv_ref, out_ref):
        shb = sh_ref[...]
        r1 = jnp.maximum(
            jnp.dot(rbf_ref[...], r1_ref[...],
                    preferred_element_type=jnp.float32) + b1_ref[...], 0.0)
        ek = (jnp.dot(shb, sk_ref[...], preferred_element_type=jnp.float32)
              * jnp.dot(r1, rk_ref[...], preferred_element_type=jnp.float32))
        ev = (jnp.dot(shb, sv_ref[...], preferred_element_type=jnp.float32)
              * jnp.dot(r1, rv_ref[...], preferred_element_type=jnp.float32))
        q = qd_ref[...]
        kk = k_ref[...]
        vv = v_ref[...]
        logit = jnp.sum(q * kk * ek, axis=1, keepdims=True) * INV_SQRT_ATT
        ex = jnp.exp(logit)
        col = lax.broadcasted_iota(jnp.int32, (_BE, DP), 1)
        out_ref[...] = ex * jnp.where(col == (DP - 1), 1.0, vv * ev)

    return pl.pallas_call(
        body,
        grid=(NE // _BE,),
        in_specs=[
            pl.BlockSpec((_BE, DP), lambda i: (i, 0)),
            pl.BlockSpec((_BE, DP), lambda i: (i, 0)),
            pl.BlockSpec((_BE, DP), lambda i: (i, 0)),
            pl.BlockSpec((_BE, DSH), lambda i: (i, 0)),
            pl.BlockSpec((_BE, DSH), lambda i: (i, 0)),
            pl.BlockSpec((DSH, DP), lambda i: (0, 0)),
            pl.BlockSpec((DSH, DP), lambda i: (0, 0)),
            pl.BlockSpec((DSH, HID), lambda i: (0, 0)),
            pl.BlockSpec((1, HID), lambda i: (0, 0)),
            pl.BlockSpec((HID, DP), lambda i: (0, 0)),
            pl.BlockSpec((HID, DP), lambda i: (0, 0)),
        ],
        out_specs=pl.BlockSpec((_BE, DP), lambda i: (i, 0)),
        out_shape=jax.ShapeDtypeStruct((NE, DP), jnp.float32),
    )(qd, ks, vs, sh, rbf, skp, svp, r1p, b1, rkp, rvp)


def _tc_finalize(s01, h, wsi, g, skip, dout):
    def body(s0_ref, s1_ref, h_ref, w_ref, g_ref, o_ref):
        num = s0_ref[...] + s1_ref[...]
        col = lax.broadcasted_iota(jnp.int32, (_BN, DP), 1)
        den = jnp.sum(jnp.where(col == (DP - 1), num, 0.0),
                      axis=1, keepdims=True) + 1e-9
        agg = jnp.where(col < dout, num, 0.0) / den
        hb = h_ref[...]
        out = agg + jnp.dot(hb, w_ref[...],
                            preferred_element_type=jnp.float32)
        if skip:
            out = out + hb
        ms = jnp.sum(out * out, axis=1, keepdims=True) / dout
        o_ref[...] = out / jnp.sqrt(ms + 1e-6) * g_ref[...]

    return pl.pallas_call(
        body,
        grid=(NN // _BN,),
        in_specs=[
            pl.BlockSpec((_BN, DP), lambda i: (i, 0)),
            pl.BlockSpec((_BN, DP), lambda i: (i + NN // _BN, 0)),
            pl.BlockSpec((_BN, DP), lambda i: (i, 0)),
            pl.BlockSpec((DP, DP), lambda i: (0, 0)),
            pl.BlockSpec((1, DP), lambda i: (0, 0)),
        ],
        out_specs=pl.BlockSpec((_BN, DP), lambda i: (i, 0)),
        out_shape=jax.ShapeDtypeStruct((NN, DP), jnp.float32),
    )(s01, s01, h, wsi, g)


def _pad2(a, r, c):
    return jnp.zeros((r, c), jnp.float32).at[:a.shape[0], :a.shape[1]].set(a)


def kernel(x, pos, params, edge_index):
    src = edge_index[0].astype(jnp.int32)
    dst = edge_index[1].astype(jnp.int32)

    pos_pad = _pad2(pos, NN, DP)
    x_pad = _pad2(x, NN, DP)
    zeros_acc = jnp.zeros((NN, DP), jnp.float32)

    pos_d, pos_s = _sc_gather_multi([pos_pad, pos_pad], [dst, src])
    sh, rbf = _tc_geom(pos_s, pos_d)

    h = x_pad
    douts = [86, 86, 86, 86, 86, 86, 40]
    skips = [False, True, True, True, True, True, False]
    for p, dout, skip in zip(params, douts, skips):
        wq = _pad2(p['Wq'], DP, DP)
        wk = _pad2(p['Wk'], DP, DP)
        wv = _pad2(p['Wv'], DP, DP)
        skp = _pad2(p['Sk'], DSH, DP)
        svp = _pad2(p['Sv'], DSH, DP)
        r1p = _pad2(p['R1'], DSH, HID)
        b1 = p['b1'].reshape(1, HID)
        rkp = _pad2(p['Rk'], HID, DP)
        rvp = _pad2(p['Rv'], HID, DP)
        wsi = _pad2(p['Wsi'], DP, DP)
        g = _pad2(p['g'].reshape(1, -1), 1, DP)

        qt, kt, vt = _tc_node(h, wq, wk, wv)
        qd, ks, vs = _sc_gather_multi([qt, kt, vt], [dst, src, src])
        payload = _tc_edge(qd, ks, vs, sh, rbf, skp, svp, r1p, b1, rkp, rvp)
        s01 = _sc_scatter_add(payload, dst, zeros_acc)
        h = _tc_finalize(s01, h, wsi, g, skip, dout)

    return h[:, :40]
